# Initial kernel scaffold; baseline (speedup 1.0000x reference)
import jax
import jax.numpy as jnp
from jax.experimental import pallas as pl


def kernel(x, edge_index, batch, goal, W1l, b1l, W1r, W2l, b2l, W2r, W3l, b3l, W3r, Wimp, bimp, Wf1, bf1, Wf2, bf2, Wf3, bf3, Wout, bout, g1, be1, g2, be2, g3, be3):
    raise NotImplementedError("write your pallas kernel here")


# speedup vs baseline: 3.2483x; 3.2483x over previous
import jax
import jax.numpy as jnp
from jax import lax
from jax.experimental import pallas as pl
from jax.experimental.pallas import tpu as pltpu
from jax.experimental.pallas import tpu_sc as plsc

N = 100000
E = 1600000
H = 64
G = 512
FB = 16
NFB = H // FB
CH = 128
NTILE = 16
NW = 2 * NTILE
NPW = 3128
NLAST = N - (NW - 1) * NPW
CAP = 131072
FLUSH = 2048
GRP = 2560
NGRP = E // GRP
STAGE = 5120
BN = 2000
NB = N // BN

_f32 = jnp.float32
_i32 = jnp.int32
_sds = jax.ShapeDtypeStruct


def _bucket_body(src1, dst1, o_bsrc, o_bdst, o_cnt,
                 sstage, dstage, sbuf, dbuf, cbuf):
    c = lax.axis_index("c")
    s = lax.axis_index("s")
    w = c * NTILE + s
    lo = w * NPW
    hi = jnp.minimum(lo + NPW, N)

    pltpu.sync_copy(src1.at[pl.ds(0, STAGE)], sstage)
    pltpu.sync_copy(src1.at[pl.ds(0, STAGE)], dstage)

    def flush_if_full(off, flushed):
        do = off >= FLUSH

        @pl.when(do)
        def _():
            dsto = w * CAP + pl.multiple_of(
                jnp.minimum(flushed, CAP - FLUSH), 8)
            pltpu.sync_copy(sstage.at[pl.ds(0, FLUSH)],
                            o_bsrc.at[pl.ds(dsto, FLUSH)])
            pltpu.sync_copy(dstage.at[pl.ds(0, FLUSH)],
                            o_bdst.at[pl.ds(dsto, FLUSH)])

            def shift(j, carry):
                a = sstage[pl.ds(FLUSH + j * 16, 16)]
                b = dstage[pl.ds(FLUSH + j * 16, 16)]
                sstage[pl.ds(j * 16, 16)] = a
                dstage[pl.ds(j * 16, 16)] = b
                return carry

            lax.fori_loop(0, (off - FLUSH + 15) // 16, shift, 0)

        return (jnp.where(do, off - FLUSH, off),
                jnp.where(do, flushed + FLUSH, flushed))

    def group(g, carry):
        off, flushed = carry
        pltpu.sync_copy(src1.at[pl.ds(g * GRP, GRP)], sbuf)
        pltpu.sync_copy(dst1.at[pl.ds(g * GRP, GRP)], dbuf)

        def batch(k, off):
            sv = sbuf[pl.ds(k * 16, 16)]
            dv = dbuf[pl.ds(k * 16, 16)]
            m = jnp.logical_and(dv >= lo, dv < hi)
            mi = m.astype(_i32)
            pos = off + plsc.cumsum(mi) - 1
            plsc.store_scatter(sstage, [pos], sv, mask=m)
            plsc.store_scatter(dstage, [pos], dv - lo, mask=m)
            return off + jnp.sum(mi)

        off = lax.fori_loop(0, GRP // 16, batch, off)
        off, flushed = flush_if_full(off, flushed)
        off, flushed = flush_if_full(off, flushed)
        return (off, flushed)

    off, flushed = lax.fori_loop(0, NGRP, group, (_i32(0), _i32(0)))

    @pl.when(off > 0)
    def _():
        dsto = w * CAP + pl.multiple_of(jnp.minimum(flushed, CAP - FLUSH), 8)
        pltpu.sync_copy(sstage.at[pl.ds(0, FLUSH)],
                        o_bsrc.at[pl.ds(dsto, FLUSH)])
        pltpu.sync_copy(dstage.at[pl.ds(0, FLUSH)],
                        o_bdst.at[pl.ds(dsto, FLUSH)])

    cbuf[...] = jnp.zeros((16,), _i32) + jnp.minimum(flushed + off, CAP)
    pltpu.sync_copy(cbuf, o_cnt.at[pl.ds(w * 16, 16)])


def _make_agg_body(nf):
    def body(*args):
        tables = args[:nf]
        bsrc, bdst, cnts, z16 = args[nf:nf + 4]
        outs = args[nf + 4:nf + 4 + nf]
        acc, sbufi, dbufi, gbuf, cbuf, sem = args[nf + 4 + nf:]
        c = lax.axis_index("c")
        s = lax.axis_index("s")
        w = c * NTILE + s
        lo = w * NPW
        pltpu.sync_copy(cnts.at[pl.ds(w * 16, 16)], cbuf)
        cnt = cbuf[...][0]
        ngr = (cnt + FLUSH - 1) // FLUSH

        for f in range(nf):
            pltpu.sync_copy(z16.at[pl.ds(0, NPW)], acc)

            def group(gi, carry):
                base = w * CAP + gi * FLUSH
                pltpu.sync_copy(bsrc.at[pl.ds(base, FLUSH)], sbufi)
                pltpu.sync_copy(bdst.at[pl.ds(base, FLUSH)], dbufi.at[pl.ds(0, FLUSH)])
                descs = [
                    pltpu.async_copy(
                        tables[f].at[sbufi.at[pl.ds(j * CH, CH)]],
                        gbuf.at[pl.ds(j * CH, CH)], sem)
                    for j in range(FLUSH // CH)
                ]
                for d in descs:
                    d.wait()
                ne = jnp.clip(cnt - gi * FLUSH, 0, FLUSH)
                nb = ne // 16

                def batch16(k, carry2):
                    dlv = dbufi[pl.ds(k * 16, 16)]
                    for t in range(16):
                        dl = dlv[t]
                        acc[dl] = acc[dl] + gbuf[k * 16 + t]
                    return carry2

                lax.fori_loop(0, nb, batch16, 0)

                def tail(e, carry2):
                    dl = dbufi[pl.ds(e, 16)][0]
                    acc[dl] = acc[dl] + gbuf[e]
                    return carry2

                lax.fori_loop(nb * 16, ne, tail, 0)
                return carry

            lax.fori_loop(0, ngr, group, 0)

            @pl.when(w < NW - 1)
            def _():
                pltpu.sync_copy(acc, outs[f].at[pl.ds(lo, NPW)])

            @pl.when(w == NW - 1)
            def _():
                pltpu.sync_copy(acc.at[pl.ds(0, NLAST)],
                                outs[f].at[pl.ds(lo, NLAST)])

    return body


def _sc_kernels():
    mesh = plsc.VectorSubcoreMesh(core_axis_name="c", subcore_axis_name="s")
    params = pltpu.CompilerParams(use_tc_tiling_on_sc=False,
                                  needs_layout_passes=False)
    bucket = pl.kernel(
        _bucket_body,
        out_type=(_sds((NW * CAP,), _i32), _sds((NW * CAP,), _i32),
                  _sds((NW * 16,), _i32)),
        mesh=mesh,
        scratch_types=[
            pltpu.VMEM((STAGE,), _i32),
            pltpu.VMEM((STAGE,), _i32),
            pltpu.VMEM((GRP,), _i32),
            pltpu.VMEM((GRP,), _i32),
            pltpu.VMEM((16,), _i32),
        ],
        compiler_params=params,
    )

    def agg_scratch():
        return [
            pltpu.VMEM((NPW, FB), _f32),
            pltpu.VMEM((FLUSH,), _i32),
            pltpu.VMEM((FLUSH + 16,), _i32),
            pltpu.VMEM((FLUSH, FB), _f32),
            pltpu.VMEM((16,), _i32),
            pltpu.SemaphoreType.DMA,
        ]

    agg1 = pl.kernel(
        _make_agg_body(1),
        out_type=_sds((N, FB), _f32),
        mesh=mesh,
        scratch_types=agg_scratch(),
        compiler_params=params,
    )
    agg4 = pl.kernel(
        _make_agg_body(NFB),
        out_type=tuple(_sds((N, FB), _f32) for _ in range(NFB)),
        mesh=mesh,
        scratch_types=agg_scratch(),
        compiler_params=params,
    )
    return bucket, agg1, agg4


def _mm1_body(ap_ref, xp_ref, wl_ref, bl_ref, wr_ref, o0, o1, o2, o3, dg_ref):
    a = ap_ref[...]
    dc = jnp.maximum(a[:, 4:5], 1.0)
    aggx = a[:, 0:4] / dc
    xb = xp_ref[:, 0:4]
    y = (jnp.dot(aggx, wl_ref[...], preferred_element_type=_f32)
         + bl_ref[...]
         + jnp.dot(xb, wr_ref[...], preferred_element_type=_f32))
    y = jnp.maximum(y, 0.0)
    o0[...] = y[:, 0:16]
    o1[...] = y[:, 16:32]
    o2[...] = y[:, 32:48]
    o3[...] = y[:, 48:64]
    dg_ref[...] = jnp.broadcast_to(dc, (BN, FB))


_mm1 = pl.pallas_call(
    _mm1_body,
    grid=(NB,),
    in_specs=[
        pl.BlockSpec((BN, FB), lambda i: (i, 0)),
        pl.BlockSpec((BN, FB), lambda i: (i, 0)),
        pl.BlockSpec((4, H), lambda i: (0, 0)),
        pl.BlockSpec((1, H), lambda i: (0, 0)),
        pl.BlockSpec((4, H), lambda i: (0, 0)),
    ],
    out_specs=[pl.BlockSpec((BN, FB), lambda i: (i, 0)) for _ in range(5)],
    out_shape=[_sds((N, FB), _f32) for _ in range(5)],
)


def _mm2_body(a0, a1, a2, a3, rd, h0, h1, h2, h3, wl_ref, bl_ref, wr_ref,
              o0, o1, o2, o3):
    r = rd[...]
    agg = jnp.concatenate(
        [a0[...] / r, a1[...] / r, a2[...] / r, a3[...] / r], axis=1)
    h = jnp.concatenate([h0[...], h1[...], h2[...], h3[...]], axis=1)
    y = (jnp.dot(agg, wl_ref[...], preferred_element_type=_f32)
         + bl_ref[...]
         + jnp.dot(h, wr_ref[...], preferred_element_type=_f32))
    y = jnp.maximum(y, 0.0)
    o0[...] = y[:, 0:16]
    o1[...] = y[:, 16:32]
    o2[...] = y[:, 32:48]
    o3[...] = y[:, 48:64]


_mm2 = pl.pallas_call(
    _mm2_body,
    grid=(NB,),
    in_specs=(
        [pl.BlockSpec((BN, FB), lambda i: (i, 0)) for _ in range(9)]
        + [pl.BlockSpec((H, H), lambda i: (0, 0)),
           pl.BlockSpec((1, H), lambda i: (0, 0)),
           pl.BlockSpec((H, H), lambda i: (0, 0))]
    ),
    out_specs=[pl.BlockSpec((BN, FB), lambda i: (i, 0)) for _ in range(4)],
    out_shape=[_sds((N, FB), _f32) for _ in range(4)],
)


def _mm3_body(a0, a1, a2, a3, rd, h0, h1, h2, h3, wl_ref, bl_ref, wr_ref,
              b3_ref, out_ref):
    i = pl.program_id(0)
    r = rd[...]
    agg = jnp.concatenate(
        [a0[...] / r, a1[...] / r, a2[...] / r, a3[...] / r], axis=1)
    h = jnp.concatenate([h0[...], h1[...], h2[...], h3[...]], axis=1)
    y = (jnp.dot(agg, wl_ref[...], preferred_element_type=_f32)
         + bl_ref[...]
         + jnp.dot(h, wr_ref[...], preferred_element_type=_f32))
    seg = b3_ref[0, 0]
    oneh = jnp.equal(lax.broadcasted_iota(jnp.int32, (G, BN), 0),
                     seg[None, :]).astype(_f32)
    pooled = jnp.dot(oneh, y, preferred_element_type=_f32,
                     precision=lax.Precision.HIGHEST)
    cnt = jnp.sum(oneh, axis=1, keepdims=True)
    chunk = jnp.concatenate(
        [pooled, cnt, jnp.zeros((G, 128 - H - 1), _f32)], axis=1)

    @pl.when(i == 0)
    def _():
        out_ref[...] = jnp.zeros_like(out_ref)

    out_ref[...] += chunk


_mm3 = pl.pallas_call(
    _mm3_body,
    grid=(NB,),
    in_specs=(
        [pl.BlockSpec((BN, FB), lambda i: (i, 0)) for _ in range(9)]
        + [pl.BlockSpec((H, H), lambda i: (0, 0)),
           pl.BlockSpec((1, H), lambda i: (0, 0)),
           pl.BlockSpec((H, H), lambda i: (0, 0)),
           pl.BlockSpec((1, 1, BN), lambda i: (i, 0, 0))]
    ),
    out_specs=pl.BlockSpec((G, 128), lambda i: (0, 0)),
    out_shape=_sds((G, 128), _f32),
)


def _bn_relu(y, gamma, beta):
    m = jnp.mean(y, axis=0)
    v = jnp.mean((y - m) ** 2, axis=0)
    return jnp.maximum((y - m) / jnp.sqrt(v + 1e-5) * gamma + beta, 0.0)


def _head_body(pp_ref, wimp_ref, bimp_ref, goal_ref, wf1_ref, bf1_ref,
               wf2_ref, bf2_ref, wf3_ref, bf3_ref, wout_ref, bout_ref,
               g1_ref, be1_ref, g2_ref, be2_ref, g3_ref, be3_ref,
               im_ref, label_ref):
    pp = pp_ref[...]
    pooled = pp[:, 0:H] / jnp.maximum(pp[:, H:H + 1], 1.0)
    imp = jnp.dot(pooled, wimp_ref[...], preferred_element_type=_f32) \
        + bimp_ref[...]
    nrm = jnp.sqrt(jnp.sum(imp * imp, axis=1, keepdims=True))
    im = imp / jnp.maximum(nrm, 1e-12)

    iota6 = lax.broadcasted_iota(jnp.int32, (G, 6), 1)
    mx = jnp.max(im, axis=1, keepdims=True)
    sel = jnp.min(jnp.where(im == mx, iota6, 6), axis=1)
    gi = (sel + 5) % 6
    oneh = jnp.equal(iota6, gi[:, None]).astype(_f32)
    gsel = jnp.dot(oneh, goal_ref[...], preferred_element_type=_f32)

    x1 = jnp.concatenate([im, gsel], axis=1)
    y = jnp.dot(x1, wf1_ref[...], preferred_element_type=_f32) + bf1_ref[...]
    y = _bn_relu(y, g1_ref[...], be1_ref[...])
    y = jnp.dot(y, wf2_ref[...], preferred_element_type=_f32) + bf2_ref[...]
    y = _bn_relu(y, g2_ref[...], be2_ref[...])
    y = jnp.dot(y, wf3_ref[...], preferred_element_type=_f32) + bf3_ref[...]
    y = _bn_relu(y, g3_ref[...], be3_ref[...])
    label_ref[...] = jnp.dot(y, wout_ref[...],
                             preferred_element_type=_f32) + bout_ref[...]
    im_ref[...] = im


_head = pl.pallas_call(
    _head_body,
    out_shape=[_sds((G, 6), _f32), _sds((G, 5), _f32)],
)


def kernel(x, edge_index, batch, goal, W1l, b1l, W1r, W2l, b2l, W2r,
           W3l, b3l, W3r, Wimp, bimp, Wf1, bf1, Wf2, bf2, Wf3, bf3,
           Wout, bout, g1, be1, g2, be2, g3, be3):
    src1 = edge_index[0]
    dst1 = edge_index[1]
    xpad = jnp.concatenate(
        [x, jnp.ones((N, 1), _f32), jnp.zeros((N, FB - 5), _f32)], axis=1)
    z16 = jnp.zeros((N, FB), _f32)

    _bucket, _agg1, _agg4 = _sc_kernels()
    bsrc, bdst, cnts = _bucket(src1, dst1)

    agg1 = _agg1(xpad, bsrc, bdst, cnts, z16)
    *h1s, dg = _mm1(agg1, xpad, W1l, b1l.reshape(1, H), W1r)

    a2 = _agg4(*h1s, bsrc, bdst, cnts, z16)
    h2s = _mm2(*a2, dg, *h1s, W2l, b2l.reshape(1, H), W2r)

    a3 = _agg4(*h2s, bsrc, bdst, cnts, z16)
    pooled_plus = _mm3(*a3, dg, *h2s, W3l, b3l.reshape(1, H), W3r,
                       batch.reshape(NB, 1, BN))

    im, label = _head(pooled_plus, Wimp, bimp.reshape(1, 6), goal,
                      Wf1, bf1.reshape(1, 512), Wf2, bf2.reshape(1, 128),
                      Wf3, bf3.reshape(1, H), Wout, bout.reshape(1, 5),
                      g1.reshape(1, 512), be1.reshape(1, 512),
                      g2.reshape(1, 128), be2.reshape(1, 128),
                      g3.reshape(1, H), be3.reshape(1, H))
    return (im, label)


# speedup vs baseline: 3.6433x; 1.1216x over previous
import jax
import jax.numpy as jnp
from jax import lax
from jax.experimental import pallas as pl
from jax.experimental.pallas import tpu as pltpu
from jax.experimental.pallas import tpu_sc as plsc

N = 100000
E = 1600000
H = 64
G = 512
FB = 16
NFB = H // FB
CH = 128
NTILE = 16
NW = 2 * NTILE
NPW = 3128
NLAST = N - (NW - 1) * NPW
CAP = 131072
FLUSH = 2048
GRP = 2560
NGRP = E // GRP
STAGE = 5120
BN = 2000
NB = N // BN

_f32 = jnp.float32
_i32 = jnp.int32
_sds = jax.ShapeDtypeStruct


def _bucket_body(src1, dst1, o_bsrc, o_bdst, o_cnt,
                 sstage, dstage, sbuf, dbuf, cbuf):
    c = lax.axis_index("c")
    s = lax.axis_index("s")
    w = c * NTILE + s
    lo = w * NPW
    hi = jnp.minimum(lo + NPW, N)

    pltpu.sync_copy(src1.at[pl.ds(0, STAGE)], sstage)
    pltpu.sync_copy(src1.at[pl.ds(0, STAGE)], dstage)

    def flush_if_full(off, flushed):
        do = off >= FLUSH

        @pl.when(do)
        def _():
            dsto = w * CAP + pl.multiple_of(
                jnp.minimum(flushed, CAP - FLUSH), 8)
            pltpu.sync_copy(sstage.at[pl.ds(0, FLUSH)],
                            o_bsrc.at[pl.ds(dsto, FLUSH)])
            pltpu.sync_copy(dstage.at[pl.ds(0, FLUSH)],
                            o_bdst.at[pl.ds(dsto, FLUSH)])

            def shift(j, carry):
                a = sstage[pl.ds(FLUSH + j * 16, 16)]
                b = dstage[pl.ds(FLUSH + j * 16, 16)]
                sstage[pl.ds(j * 16, 16)] = a
                dstage[pl.ds(j * 16, 16)] = b
                return carry

            lax.fori_loop(0, (off - FLUSH + 15) // 16, shift, 0)

        return (jnp.where(do, off - FLUSH, off),
                jnp.where(do, flushed + FLUSH, flushed))

    def group(g, carry):
        off, flushed = carry
        pltpu.sync_copy(src1.at[pl.ds(g * GRP, GRP)], sbuf)
        pltpu.sync_copy(dst1.at[pl.ds(g * GRP, GRP)], dbuf)

        def batch(k, off):
            for u in range(8):
                b0 = k * 128 + u * 16
                sv = sbuf[pl.ds(b0, 16)]
                dv = dbuf[pl.ds(b0, 16)]
                m = jnp.logical_and(dv >= lo, dv < hi)
                mi = m.astype(_i32)
                pc = plsc.cumsum(mi)
                pos = off + pc - 1
                plsc.store_scatter(sstage, [pos], sv, mask=m)
                plsc.store_scatter(dstage, [pos], dv - lo, mask=m)
                off = off + pc[15]
            return off

        off = lax.fori_loop(0, GRP // 128, batch, off)
        off, flushed = flush_if_full(off, flushed)
        off, flushed = flush_if_full(off, flushed)
        return (off, flushed)

    off, flushed = lax.fori_loop(0, NGRP, group, (_i32(0), _i32(0)))

    @pl.when(off > 0)
    def _():
        dsto = w * CAP + pl.multiple_of(jnp.minimum(flushed, CAP - FLUSH), 8)
        pltpu.sync_copy(sstage.at[pl.ds(0, FLUSH)],
                        o_bsrc.at[pl.ds(dsto, FLUSH)])
        pltpu.sync_copy(dstage.at[pl.ds(0, FLUSH)],
                        o_bdst.at[pl.ds(dsto, FLUSH)])

    cbuf[...] = jnp.zeros((16,), _i32) + jnp.minimum(flushed + off, CAP)
    pltpu.sync_copy(cbuf, o_cnt.at[pl.ds(w * 16, 16)])


def _make_agg_body(nf):
    def body(*args):
        tables = args[:nf]
        bsrc, bdst, cnts, z16 = args[nf:nf + 4]
        outs = args[nf + 4:nf + 4 + nf]
        acc, sbufi, dbufi, gbuf, cbuf, sem = args[nf + 4 + nf:]
        c = lax.axis_index("c")
        s = lax.axis_index("s")
        w = c * NTILE + s
        lo = w * NPW
        pltpu.sync_copy(cnts.at[pl.ds(w * 16, 16)], cbuf)
        cnt = cbuf[...][0]
        ngr = (cnt + FLUSH - 1) // FLUSH

        for f in range(nf):
            pltpu.sync_copy(z16.at[pl.ds(0, NPW)], acc)

            def group(gi, carry):
                base = w * CAP + gi * FLUSH
                pltpu.sync_copy(bsrc.at[pl.ds(base, FLUSH)], sbufi)
                pltpu.sync_copy(bdst.at[pl.ds(base, FLUSH)], dbufi.at[pl.ds(0, FLUSH)])
                descs = [
                    pltpu.async_copy(
                        tables[f].at[sbufi.at[pl.ds(j * CH, CH)]],
                        gbuf.at[pl.ds(j * CH, CH)], sem)
                    for j in range(FLUSH // CH)
                ]
                for d in descs:
                    d.wait()
                ne = jnp.clip(cnt - gi * FLUSH, 0, FLUSH)
                nb = ne // 16

                def batch16(k, carry2):
                    dlv = dbufi[pl.ds(k * 16, 16)]
                    for t in range(16):
                        dl = dlv[t]
                        plsc.addupdate(acc.at[dl], gbuf[k * 16 + t])
                    return carry2

                lax.fori_loop(0, nb, batch16, 0)

                def tail(e, carry2):
                    dl = dbufi[pl.ds(e, 16)][0]
                    plsc.addupdate(acc.at[dl], gbuf[e])
                    return carry2

                lax.fori_loop(nb * 16, ne, tail, 0)
                return carry

            lax.fori_loop(0, ngr, group, 0)

            @pl.when(w < NW - 1)
            def _():
                pltpu.sync_copy(acc, outs[f].at[pl.ds(lo, NPW)])

            @pl.when(w == NW - 1)
            def _():
                pltpu.sync_copy(acc.at[pl.ds(0, NLAST)],
                                outs[f].at[pl.ds(lo, NLAST)])

    return body


def _sc_kernels():
    mesh = plsc.VectorSubcoreMesh(core_axis_name="c", subcore_axis_name="s")
    params = pltpu.CompilerParams(use_tc_tiling_on_sc=False,
                                  needs_layout_passes=False)
    bucket = pl.kernel(
        _bucket_body,
        out_type=(_sds((NW * CAP,), _i32), _sds((NW * CAP,), _i32),
                  _sds((NW * 16,), _i32)),
        mesh=mesh,
        scratch_types=[
            pltpu.VMEM((STAGE,), _i32),
            pltpu.VMEM((STAGE,), _i32),
            pltpu.VMEM((GRP,), _i32),
            pltpu.VMEM((GRP,), _i32),
            pltpu.VMEM((16,), _i32),
        ],
        compiler_params=params,
    )

    def agg_scratch():
        return [
            pltpu.VMEM((NPW, FB), _f32),
            pltpu.VMEM((FLUSH,), _i32),
            pltpu.VMEM((FLUSH + 16,), _i32),
            pltpu.VMEM((FLUSH, FB), _f32),
            pltpu.VMEM((16,), _i32),
            pltpu.SemaphoreType.DMA,
        ]

    agg1 = pl.kernel(
        _make_agg_body(1),
        out_type=_sds((N, FB), _f32),
        mesh=mesh,
        scratch_types=agg_scratch(),
        compiler_params=params,
    )
    agg4 = pl.kernel(
        _make_agg_body(NFB),
        out_type=tuple(_sds((N, FB), _f32) for _ in range(NFB)),
        mesh=mesh,
        scratch_types=agg_scratch(),
        compiler_params=params,
    )
    return bucket, agg1, agg4


def _mm1_body(ap_ref, xp_ref, wl_ref, bl_ref, wr_ref, o0, o1, o2, o3, dg_ref):
    a = ap_ref[...]
    dc = jnp.maximum(a[:, 4:5], 1.0)
    aggx = a[:, 0:4] / dc
    xb = xp_ref[:, 0:4]
    y = (jnp.dot(aggx, wl_ref[...], preferred_element_type=_f32)
         + bl_ref[...]
         + jnp.dot(xb, wr_ref[...], preferred_element_type=_f32))
    y = jnp.maximum(y, 0.0)
    o0[...] = y[:, 0:16]
    o1[...] = y[:, 16:32]
    o2[...] = y[:, 32:48]
    o3[...] = y[:, 48:64]
    dg_ref[...] = jnp.broadcast_to(dc, (BN, FB))


_mm1 = pl.pallas_call(
    _mm1_body,
    grid=(NB,),
    in_specs=[
        pl.BlockSpec((BN, FB), lambda i: (i, 0)),
        pl.BlockSpec((BN, FB), lambda i: (i, 0)),
        pl.BlockSpec((4, H), lambda i: (0, 0)),
        pl.BlockSpec((1, H), lambda i: (0, 0)),
        pl.BlockSpec((4, H), lambda i: (0, 0)),
    ],
    out_specs=[pl.BlockSpec((BN, FB), lambda i: (i, 0)) for _ in range(5)],
    out_shape=[_sds((N, FB), _f32) for _ in range(5)],
)


def _mm2_body(a0, a1, a2, a3, rd, h0, h1, h2, h3, wl_ref, bl_ref, wr_ref,
              o0, o1, o2, o3):
    r = rd[...]
    agg = jnp.concatenate(
        [a0[...] / r, a1[...] / r, a2[...] / r, a3[...] / r], axis=1)
    h = jnp.concatenate([h0[...], h1[...], h2[...], h3[...]], axis=1)
    y = (jnp.dot(agg, wl_ref[...], preferred_element_type=_f32)
         + bl_ref[...]
         + jnp.dot(h, wr_ref[...], preferred_element_type=_f32))
    y = jnp.maximum(y, 0.0)
    o0[...] = y[:, 0:16]
    o1[...] = y[:, 16:32]
    o2[...] = y[:, 32:48]
    o3[...] = y[:, 48:64]


_mm2 = pl.pallas_call(
    _mm2_body,
    grid=(NB,),
    in_specs=(
        [pl.BlockSpec((BN, FB), lambda i: (i, 0)) for _ in range(9)]
        + [pl.BlockSpec((H, H), lambda i: (0, 0)),
           pl.BlockSpec((1, H), lambda i: (0, 0)),
           pl.BlockSpec((H, H), lambda i: (0, 0))]
    ),
    out_specs=[pl.BlockSpec((BN, FB), lambda i: (i, 0)) for _ in range(4)],
    out_shape=[_sds((N, FB), _f32) for _ in range(4)],
)


def _mm3_body(a0, a1, a2, a3, rd, h0, h1, h2, h3, wl_ref, bl_ref, wr_ref,
              b3_ref, out_ref):
    i = pl.program_id(0)
    r = rd[...]
    agg = jnp.concatenate(
        [a0[...] / r, a1[...] / r, a2[...] / r, a3[...] / r], axis=1)
    h = jnp.concatenate([h0[...], h1[...], h2[...], h3[...]], axis=1)
    y = (jnp.dot(agg, wl_ref[...], preferred_element_type=_f32)
         + bl_ref[...]
         + jnp.dot(h, wr_ref[...], preferred_element_type=_f32))
    seg = b3_ref[0, 0]
    oneh = jnp.equal(lax.broadcasted_iota(jnp.int32, (G, BN), 0),
                     seg[None, :]).astype(_f32)
    pooled = jnp.dot(oneh, y, preferred_element_type=_f32,
                     precision=lax.Precision.HIGHEST)
    cnt = jnp.sum(oneh, axis=1, keepdims=True)
    chunk = jnp.concatenate(
        [pooled, cnt, jnp.zeros((G, 128 - H - 1), _f32)], axis=1)

    @pl.when(i == 0)
    def _():
        out_ref[...] = jnp.zeros_like(out_ref)

    out_ref[...] += chunk


_mm3 = pl.pallas_call(
    _mm3_body,
    grid=(NB,),
    in_specs=(
        [pl.BlockSpec((BN, FB), lambda i: (i, 0)) for _ in range(9)]
        + [pl.BlockSpec((H, H), lambda i: (0, 0)),
           pl.BlockSpec((1, H), lambda i: (0, 0)),
           pl.BlockSpec((H, H), lambda i: (0, 0)),
           pl.BlockSpec((1, 1, BN), lambda i: (i, 0, 0))]
    ),
    out_specs=pl.BlockSpec((G, 128), lambda i: (0, 0)),
    out_shape=_sds((G, 128), _f32),
)


def _bn_relu(y, gamma, beta):
    m = jnp.mean(y, axis=0)
    v = jnp.mean((y - m) ** 2, axis=0)
    return jnp.maximum((y - m) / jnp.sqrt(v + 1e-5) * gamma + beta, 0.0)


def _head_body(pp_ref, wimp_ref, bimp_ref, goal_ref, wf1_ref, bf1_ref,
               wf2_ref, bf2_ref, wf3_ref, bf3_ref, wout_ref, bout_ref,
               g1_ref, be1_ref, g2_ref, be2_ref, g3_ref, be3_ref,
               im_ref, label_ref):
    pp = pp_ref[...]
    pooled = pp[:, 0:H] / jnp.maximum(pp[:, H:H + 1], 1.0)
    imp = jnp.dot(pooled, wimp_ref[...], preferred_element_type=_f32) \
        + bimp_ref[...]
    nrm = jnp.sqrt(jnp.sum(imp * imp, axis=1, keepdims=True))
    im = imp / jnp.maximum(nrm, 1e-12)

    iota6 = lax.broadcasted_iota(jnp.int32, (G, 6), 1)
    mx = jnp.max(im, axis=1, keepdims=True)
    sel = jnp.min(jnp.where(im == mx, iota6, 6), axis=1)
    gi = (sel + 5) % 6
    oneh = jnp.equal(iota6, gi[:, None]).astype(_f32)
    gsel = jnp.dot(oneh, goal_ref[...], preferred_element_type=_f32)

    x1 = jnp.concatenate([im, gsel], axis=1)
    y = jnp.dot(x1, wf1_ref[...], preferred_element_type=_f32) + bf1_ref[...]
    y = _bn_relu(y, g1_ref[...], be1_ref[...])
    y = jnp.dot(y, wf2_ref[...], preferred_element_type=_f32) + bf2_ref[...]
    y = _bn_relu(y, g2_ref[...], be2_ref[...])
    y = jnp.dot(y, wf3_ref[...], preferred_element_type=_f32) + bf3_ref[...]
    y = _bn_relu(y, g3_ref[...], be3_ref[...])
    label_ref[...] = jnp.dot(y, wout_ref[...],
                             preferred_element_type=_f32) + bout_ref[...]
    im_ref[...] = im


_head = pl.pallas_call(
    _head_body,
    out_shape=[_sds((G, 6), _f32), _sds((G, 5), _f32)],
)


def kernel(x, edge_index, batch, goal, W1l, b1l, W1r, W2l, b2l, W2r,
           W3l, b3l, W3r, Wimp, bimp, Wf1, bf1, Wf2, bf2, Wf3, bf3,
           Wout, bout, g1, be1, g2, be2, g3, be3):
    src1 = edge_index[0]
    dst1 = edge_index[1]
    xpad = jnp.concatenate(
        [x, jnp.ones((N, 1), _f32), jnp.zeros((N, FB - 5), _f32)], axis=1)
    z16 = jnp.zeros((N, FB), _f32)

    _bucket, _agg1, _agg4 = _sc_kernels()
    bsrc, bdst, cnts = _bucket(src1, dst1)

    agg1 = _agg1(xpad, bsrc, bdst, cnts, z16)
    *h1s, dg = _mm1(agg1, xpad, W1l, b1l.reshape(1, H), W1r)

    a2 = _agg4(*h1s, bsrc, bdst, cnts, z16)
    h2s = _mm2(*a2, dg, *h1s, W2l, b2l.reshape(1, H), W2r)

    a3 = _agg4(*h2s, bsrc, bdst, cnts, z16)
    pooled_plus = _mm3(*a3, dg, *h2s, W3l, b3l.reshape(1, H), W3r,
                       batch.reshape(NB, 1, BN))

    im, label = _head(pooled_plus, Wimp, bimp.reshape(1, 6), goal,
                      Wf1, bf1.reshape(1, 512), Wf2, bf2.reshape(1, 128),
                      Wf3, bf3.reshape(1, H), Wout, bout.reshape(1, 5),
                      g1.reshape(1, 512), be1.reshape(1, 512),
                      g2.reshape(1, 128), be2.reshape(1, 128),
                      g3.reshape(1, H), be3.reshape(1, H))
    return (im, label)


# speedup vs baseline: 4.0453x; 1.1104x over previous
import jax
import jax.numpy as jnp
from jax import lax
from jax.experimental import pallas as pl
from jax.experimental.pallas import tpu as pltpu
from jax.experimental.pallas import tpu_sc as plsc

N = 100000
E = 1600000
H = 64
G = 512
FB = 16
NFB = H // FB
CH = 128
NTILE = 16
NW = 2 * NTILE
NPW = 3128
NLAST = N - (NW - 1) * NPW
CAP = 131072
FLUSH = 2048
GRP = 2560
NGRP = E // GRP
STAGE = 5120
BN = 2000
NB = N // BN

_f32 = jnp.float32
_i32 = jnp.int32
_sds = jax.ShapeDtypeStruct


def _bucket_body(src1, dst1, o_bsrc, o_bdst, o_cnt,
                 sstage, dstage, sbuf, dbuf, cbuf):
    c = lax.axis_index("c")
    s = lax.axis_index("s")
    w = c * NTILE + s
    lo = w * NPW
    hi = jnp.minimum(lo + NPW, N)

    pltpu.sync_copy(src1.at[pl.ds(0, STAGE)], sstage)
    pltpu.sync_copy(src1.at[pl.ds(0, STAGE)], dstage)

    def flush_if_full(off, flushed):
        do = off >= FLUSH

        @pl.when(do)
        def _():
            dsto = w * CAP + pl.multiple_of(
                jnp.minimum(flushed, CAP - FLUSH), 8)
            pltpu.sync_copy(sstage.at[pl.ds(0, FLUSH)],
                            o_bsrc.at[pl.ds(dsto, FLUSH)])
            pltpu.sync_copy(dstage.at[pl.ds(0, FLUSH)],
                            o_bdst.at[pl.ds(dsto, FLUSH)])

            def shift(j, carry):
                a = sstage[pl.ds(FLUSH + j * 16, 16)]
                b = dstage[pl.ds(FLUSH + j * 16, 16)]
                sstage[pl.ds(j * 16, 16)] = a
                dstage[pl.ds(j * 16, 16)] = b
                return carry

            lax.fori_loop(0, (off - FLUSH + 15) // 16, shift, 0)

        return (jnp.where(do, off - FLUSH, off),
                jnp.where(do, flushed + FLUSH, flushed))

    def group(g, carry):
        off, flushed = carry
        pltpu.sync_copy(src1.at[pl.ds(g * GRP, GRP)], sbuf)
        pltpu.sync_copy(dst1.at[pl.ds(g * GRP, GRP)], dbuf)

        def batch(k, off):
            for u in range(8):
                b0 = k * 128 + u * 16
                sv = sbuf[pl.ds(b0, 16)]
                dv = dbuf[pl.ds(b0, 16)]
                m = jnp.logical_and(dv >= lo, dv < hi)
                mi = m.astype(_i32)
                pc = plsc.cumsum(mi)
                pos = off + pc - 1
                plsc.store_scatter(sstage, [pos], sv, mask=m)
                plsc.store_scatter(dstage, [pos], dv - lo, mask=m)
                off = off + pc[15]
            return off

        off = lax.fori_loop(0, GRP // 128, batch, off)
        off, flushed = flush_if_full(off, flushed)
        off, flushed = flush_if_full(off, flushed)
        return (off, flushed)

    off, flushed = lax.fori_loop(0, NGRP, group, (_i32(0), _i32(0)))

    @pl.when(off > 0)
    def _():
        dsto = w * CAP + pl.multiple_of(jnp.minimum(flushed, CAP - FLUSH), 8)
        pltpu.sync_copy(sstage.at[pl.ds(0, FLUSH)],
                        o_bsrc.at[pl.ds(dsto, FLUSH)])
        pltpu.sync_copy(dstage.at[pl.ds(0, FLUSH)],
                        o_bdst.at[pl.ds(dsto, FLUSH)])

    cbuf[...] = jnp.zeros((16,), _i32) + jnp.minimum(flushed + off, CAP)
    pltpu.sync_copy(cbuf, o_cnt.at[pl.ds(w * 16, 16)])


def _make_agg_body(nf):
    def body(*args):
        tables = args[:nf]
        bsrc, bdst, cnts, z16 = args[nf:nf + 4]
        outs = args[nf + 4:nf + 4 + nf]
        (acc, sbufa, dbufa, gbufa, sbufb, dbufb, gbufb, cbuf,
         sema, semb) = args[nf + 4 + nf:]
        c = lax.axis_index("c")
        s = lax.axis_index("s")
        w = c * NTILE + s
        lo = w * NPW
        pltpu.sync_copy(cnts.at[pl.ds(w * 16, 16)], cbuf)
        cnt = cbuf[...][0]
        ngr = (cnt + FLUSH - 1) // FLUSH

        for f in range(nf):
            tref = tables[f]

            def stage_fire(g, sb, db, gb, sem):
                base = w * CAP + g * FLUSH
                pltpu.sync_copy(bsrc.at[pl.ds(base, FLUSH)], sb)
                pltpu.sync_copy(bdst.at[pl.ds(base, FLUSH)],
                                db.at[pl.ds(0, FLUSH)])
                for j in range(FLUSH // CH):
                    pltpu.async_copy(tref.at[sb.at[pl.ds(j * CH, CH)]],
                                     gb.at[pl.ds(j * CH, CH)], sem)

            def wait_gb(gb, sem):
                pltpu.make_async_copy(tref.at[pl.ds(0, FLUSH)], gb,
                                      sem).wait()

            def process(g, db, gb):
                ne = jnp.clip(cnt - g * FLUSH, 0, FLUSH)
                nb = ne // 16

                def batch16(k, carry2):
                    dlv = db[pl.ds(k * 16, 16)]
                    for t in range(16):
                        dl = dlv[t]
                        plsc.addupdate(acc.at[dl], gb[k * 16 + t])
                    return carry2

                lax.fori_loop(0, nb, batch16, 0)

                def tail(e, carry2):
                    dl = db[pl.ds(e, 16)][0]
                    plsc.addupdate(acc.at[dl], gb[e])
                    return carry2

                lax.fori_loop(nb * 16, ne, tail, 0)

            pltpu.sync_copy(z16.at[pl.ds(0, NPW)], acc)

            @pl.when(ngr > 0)
            def _():
                stage_fire(0, sbufa, dbufa, gbufa, sema)

            def pair(gg, carry):
                g0 = gg * 2
                g1 = g0 + 1

                @pl.when(g0 < ngr)
                def _():
                    wait_gb(gbufa, sema)

                    @pl.when(g1 < ngr)
                    def _():
                        stage_fire(g1, sbufb, dbufb, gbufb, semb)

                    process(g0, dbufa, gbufa)

                @pl.when(g1 < ngr)
                def _():
                    wait_gb(gbufb, semb)

                    @pl.when(g1 + 1 < ngr)
                    def _():
                        stage_fire(g1 + 1, sbufa, dbufa, gbufa, sema)

                    process(g1, dbufb, gbufb)

                return carry

            lax.fori_loop(0, (ngr + 1) // 2, pair, 0)

            @pl.when(w < NW - 1)
            def _():
                pltpu.sync_copy(acc, outs[f].at[pl.ds(lo, NPW)])

            @pl.when(w == NW - 1)
            def _():
                pltpu.sync_copy(acc.at[pl.ds(0, NLAST)],
                                outs[f].at[pl.ds(lo, NLAST)])

    return body


def _sc_kernels():
    mesh = plsc.VectorSubcoreMesh(core_axis_name="c", subcore_axis_name="s")
    params = pltpu.CompilerParams(use_tc_tiling_on_sc=False,
                                  needs_layout_passes=False)
    bucket = pl.kernel(
        _bucket_body,
        out_type=(_sds((NW * CAP,), _i32), _sds((NW * CAP,), _i32),
                  _sds((NW * 16,), _i32)),
        mesh=mesh,
        scratch_types=[
            pltpu.VMEM((STAGE,), _i32),
            pltpu.VMEM((STAGE,), _i32),
            pltpu.VMEM((GRP,), _i32),
            pltpu.VMEM((GRP,), _i32),
            pltpu.VMEM((16,), _i32),
        ],
        compiler_params=params,
    )

    def agg_scratch():
        return [
            pltpu.VMEM((NPW, FB), _f32),
            pltpu.VMEM((FLUSH,), _i32),
            pltpu.VMEM((FLUSH + 16,), _i32),
            pltpu.VMEM((FLUSH, FB), _f32),
            pltpu.VMEM((FLUSH,), _i32),
            pltpu.VMEM((FLUSH + 16,), _i32),
            pltpu.VMEM((FLUSH, FB), _f32),
            pltpu.VMEM((16,), _i32),
            pltpu.SemaphoreType.DMA,
            pltpu.SemaphoreType.DMA,
        ]

    agg1 = pl.kernel(
        _make_agg_body(1),
        out_type=_sds((N, FB), _f32),
        mesh=mesh,
        scratch_types=agg_scratch(),
        compiler_params=params,
    )
    agg4 = pl.kernel(
        _make_agg_body(NFB),
        out_type=tuple(_sds((N, FB), _f32) for _ in range(NFB)),
        mesh=mesh,
        scratch_types=agg_scratch(),
        compiler_params=params,
    )
    return bucket, agg1, agg4


def _mm1_body(ap_ref, xp_ref, wl_ref, bl_ref, wr_ref, o0, o1, o2, o3, dg_ref):
    a = ap_ref[...]
    dc = jnp.maximum(a[:, 4:5], 1.0)
    aggx = a[:, 0:4] / dc
    xb = xp_ref[:, 0:4]
    y = (jnp.dot(aggx, wl_ref[...], preferred_element_type=_f32)
         + bl_ref[...]
         + jnp.dot(xb, wr_ref[...], preferred_element_type=_f32))
    y = jnp.maximum(y, 0.0)
    o0[...] = y[:, 0:16]
    o1[...] = y[:, 16:32]
    o2[...] = y[:, 32:48]
    o3[...] = y[:, 48:64]
    dg_ref[...] = jnp.broadcast_to(dc, (BN, FB))


_mm1 = pl.pallas_call(
    _mm1_body,
    grid=(NB,),
    in_specs=[
        pl.BlockSpec((BN, FB), lambda i: (i, 0)),
        pl.BlockSpec((BN, FB), lambda i: (i, 0)),
        pl.BlockSpec((4, H), lambda i: (0, 0)),
        pl.BlockSpec((1, H), lambda i: (0, 0)),
        pl.BlockSpec((4, H), lambda i: (0, 0)),
    ],
    out_specs=[pl.BlockSpec((BN, FB), lambda i: (i, 0)) for _ in range(5)],
    out_shape=[_sds((N, FB), _f32) for _ in range(5)],
)


def _mm2_body(a0, a1, a2, a3, rd, h0, h1, h2, h3, wl_ref, bl_ref, wr_ref,
              o0, o1, o2, o3):
    r = rd[...]
    agg = jnp.concatenate(
        [a0[...] / r, a1[...] / r, a2[...] / r, a3[...] / r], axis=1)
    h = jnp.concatenate([h0[...], h1[...], h2[...], h3[...]], axis=1)
    y = (jnp.dot(agg, wl_ref[...], preferred_element_type=_f32)
         + bl_ref[...]
         + jnp.dot(h, wr_ref[...], preferred_element_type=_f32))
    y = jnp.maximum(y, 0.0)
    o0[...] = y[:, 0:16]
    o1[...] = y[:, 16:32]
    o2[...] = y[:, 32:48]
    o3[...] = y[:, 48:64]


_mm2 = pl.pallas_call(
    _mm2_body,
    grid=(NB,),
    in_specs=(
        [pl.BlockSpec((BN, FB), lambda i: (i, 0)) for _ in range(9)]
        + [pl.BlockSpec((H, H), lambda i: (0, 0)),
           pl.BlockSpec((1, H), lambda i: (0, 0)),
           pl.BlockSpec((H, H), lambda i: (0, 0))]
    ),
    out_specs=[pl.BlockSpec((BN, FB), lambda i: (i, 0)) for _ in range(4)],
    out_shape=[_sds((N, FB), _f32) for _ in range(4)],
)


def _mm3_body(a0, a1, a2, a3, rd, h0, h1, h2, h3, wl_ref, bl_ref, wr_ref,
              b3_ref, out_ref):
    i = pl.program_id(0)
    r = rd[...]
    agg = jnp.concatenate(
        [a0[...] / r, a1[...] / r, a2[...] / r, a3[...] / r], axis=1)
    h = jnp.concatenate([h0[...], h1[...], h2[...], h3[...]], axis=1)
    y = (jnp.dot(agg, wl_ref[...], preferred_element_type=_f32)
         + bl_ref[...]
         + jnp.dot(h, wr_ref[...], preferred_element_type=_f32))
    seg = b3_ref[0, 0]
    oneh = jnp.equal(lax.broadcasted_iota(jnp.int32, (G, BN), 0),
                     seg[None, :]).astype(_f32)
    pooled = jnp.dot(oneh, y, preferred_element_type=_f32,
                     precision=lax.Precision.HIGHEST)
    cnt = jnp.sum(oneh, axis=1, keepdims=True)
    chunk = jnp.concatenate(
        [pooled, cnt, jnp.zeros((G, 128 - H - 1), _f32)], axis=1)

    @pl.when(i == 0)
    def _():
        out_ref[...] = jnp.zeros_like(out_ref)

    out_ref[...] += chunk


_mm3 = pl.pallas_call(
    _mm3_body,
    grid=(NB,),
    in_specs=(
        [pl.BlockSpec((BN, FB), lambda i: (i, 0)) for _ in range(9)]
        + [pl.BlockSpec((H, H), lambda i: (0, 0)),
           pl.BlockSpec((1, H), lambda i: (0, 0)),
           pl.BlockSpec((H, H), lambda i: (0, 0)),
           pl.BlockSpec((1, 1, BN), lambda i: (i, 0, 0))]
    ),
    out_specs=pl.BlockSpec((G, 128), lambda i: (0, 0)),
    out_shape=_sds((G, 128), _f32),
)


def _bn_relu(y, gamma, beta):
    m = jnp.mean(y, axis=0)
    v = jnp.mean((y - m) ** 2, axis=0)
    return jnp.maximum((y - m) / jnp.sqrt(v + 1e-5) * gamma + beta, 0.0)


def _head_body(pp_ref, wimp_ref, bimp_ref, goal_ref, wf1_ref, bf1_ref,
               wf2_ref, bf2_ref, wf3_ref, bf3_ref, wout_ref, bout_ref,
               g1_ref, be1_ref, g2_ref, be2_ref, g3_ref, be3_ref,
               im_ref, label_ref):
    pp = pp_ref[...]
    pooled = pp[:, 0:H] / jnp.maximum(pp[:, H:H + 1], 1.0)
    imp = jnp.dot(pooled, wimp_ref[...], preferred_element_type=_f32) \
        + bimp_ref[...]
    nrm = jnp.sqrt(jnp.sum(imp * imp, axis=1, keepdims=True))
    im = imp / jnp.maximum(nrm, 1e-12)

    iota6 = lax.broadcasted_iota(jnp.int32, (G, 6), 1)
    mx = jnp.max(im, axis=1, keepdims=True)
    sel = jnp.min(jnp.where(im == mx, iota6, 6), axis=1)
    gi = (sel + 5) % 6
    oneh = jnp.equal(iota6, gi[:, None]).astype(_f32)
    gsel = jnp.dot(oneh, goal_ref[...], preferred_element_type=_f32)

    x1 = jnp.concatenate([im, gsel], axis=1)
    y = jnp.dot(x1, wf1_ref[...], preferred_element_type=_f32) + bf1_ref[...]
    y = _bn_relu(y, g1_ref[...], be1_ref[...])
    y = jnp.dot(y, wf2_ref[...], preferred_element_type=_f32) + bf2_ref[...]
    y = _bn_relu(y, g2_ref[...], be2_ref[...])
    y = jnp.dot(y, wf3_ref[...], preferred_element_type=_f32) + bf3_ref[...]
    y = _bn_relu(y, g3_ref[...], be3_ref[...])
    label_ref[...] = jnp.dot(y, wout_ref[...],
                             preferred_element_type=_f32) + bout_ref[...]
    im_ref[...] = im


_head = pl.pallas_call(
    _head_body,
    out_shape=[_sds((G, 6), _f32), _sds((G, 5), _f32)],
)


def kernel(x, edge_index, batch, goal, W1l, b1l, W1r, W2l, b2l, W2r,
           W3l, b3l, W3r, Wimp, bimp, Wf1, bf1, Wf2, bf2, Wf3, bf3,
           Wout, bout, g1, be1, g2, be2, g3, be3):
    src1 = edge_index[0]
    dst1 = edge_index[1]
    xpad = jnp.concatenate(
        [x, jnp.ones((N, 1), _f32), jnp.zeros((N, FB - 5), _f32)], axis=1)
    z16 = jnp.zeros((N, FB), _f32)

    _bucket, _agg1, _agg4 = _sc_kernels()
    bsrc, bdst, cnts = _bucket(src1, dst1)

    agg1 = _agg1(xpad, bsrc, bdst, cnts, z16)
    *h1s, dg = _mm1(agg1, xpad, W1l, b1l.reshape(1, H), W1r)

    a2 = _agg4(*h1s, bsrc, bdst, cnts, z16)
    h2s = _mm2(*a2, dg, *h1s, W2l, b2l.reshape(1, H), W2r)

    a3 = _agg4(*h2s, bsrc, bdst, cnts, z16)
    pooled_plus = _mm3(*a3, dg, *h2s, W3l, b3l.reshape(1, H), W3r,
                       batch.reshape(NB, 1, BN))

    im, label = _head(pooled_plus, Wimp, bimp.reshape(1, 6), goal,
                      Wf1, bf1.reshape(1, 512), Wf2, bf2.reshape(1, 128),
                      Wf3, bf3.reshape(1, H), Wout, bout.reshape(1, 5),
                      g1.reshape(1, 512), be1.reshape(1, 512),
                      g2.reshape(1, 128), be2.reshape(1, 128),
                      g3.reshape(1, H), be3.reshape(1, H))
    return (im, label)


# speedup vs baseline: 4.4948x; 1.1111x over previous
import jax
import jax.numpy as jnp
from jax import lax
from jax.experimental import pallas as pl
from jax.experimental.pallas import tpu as pltpu
from jax.experimental.pallas import tpu_sc as plsc

N = 100000
E = 1600000
H = 64
G = 512
FB = 16
NFB = H // FB
CH = 128
NTILE = 16
NW = 2 * NTILE
NPW = 3128
NLAST = N - (NW - 1) * NPW
CAP = 131072
FLUSH = 2048
GRP = 12800
NGRP = E // GRP
STAGE = 17408
BN = 2000
NB = N // BN

_f32 = jnp.float32
_i32 = jnp.int32
_sds = jax.ShapeDtypeStruct


def _bucket_body(src1, dst1, o_bsrc, o_bdst, o_cnt,
                 sstage, dstage, sbuf, dbuf, cbuf):
    c = lax.axis_index("c")
    s = lax.axis_index("s")
    w = c * NTILE + s
    lo = w * NPW
    hi = jnp.minimum(lo + NPW, N)

    pltpu.sync_copy(src1.at[pl.ds(0, STAGE)], sstage)
    pltpu.sync_copy(src1.at[pl.ds(0, STAGE)], dstage)

    def flush_if_full(off, flushed):
        do = off >= FLUSH

        @pl.when(do)
        def _():
            dsto = w * CAP + pl.multiple_of(
                jnp.minimum(flushed, CAP - FLUSH), 8)
            pltpu.sync_copy(sstage.at[pl.ds(0, FLUSH)],
                            o_bsrc.at[pl.ds(dsto, FLUSH)])
            pltpu.sync_copy(dstage.at[pl.ds(0, FLUSH)],
                            o_bdst.at[pl.ds(dsto, FLUSH)])

            def shift(j, carry):
                a = sstage[pl.ds(FLUSH + j * 16, 16)]
                b = dstage[pl.ds(FLUSH + j * 16, 16)]
                sstage[pl.ds(j * 16, 16)] = a
                dstage[pl.ds(j * 16, 16)] = b
                return carry

            lax.fori_loop(0, (off - FLUSH + 15) // 16, shift, 0)

        return (jnp.where(do, off - FLUSH, off),
                jnp.where(do, flushed + FLUSH, flushed))

    def group(g, carry):
        off, flushed = carry
        pltpu.sync_copy(src1.at[pl.ds(g * GRP, GRP)], sbuf)
        pltpu.sync_copy(dst1.at[pl.ds(g * GRP, GRP)], dbuf)

        def batch(k, off):
            for u in range(8):
                b0 = k * 128 + u * 16
                sv = sbuf[pl.ds(b0, 16)]
                dv = dbuf[pl.ds(b0, 16)]
                m = jnp.logical_and(dv >= lo, dv < hi)
                mi = m.astype(_i32)
                pc = plsc.cumsum(mi)
                pos = off + pc - 1
                plsc.store_scatter(sstage, [pos], sv, mask=m)
                plsc.store_scatter(dstage, [pos], dv - lo, mask=m)
                off = off + pc[15]
            return off

        off = lax.fori_loop(0, GRP // 128, batch, off)
        for _ in range(GRP // FLUSH + 1):
            off, flushed = flush_if_full(off, flushed)
        return (off, flushed)

    off, flushed = lax.fori_loop(0, NGRP, group, (_i32(0), _i32(0)))

    @pl.when(off > 0)
    def _():
        dsto = w * CAP + pl.multiple_of(jnp.minimum(flushed, CAP - FLUSH), 8)
        pltpu.sync_copy(sstage.at[pl.ds(0, FLUSH)],
                        o_bsrc.at[pl.ds(dsto, FLUSH)])
        pltpu.sync_copy(dstage.at[pl.ds(0, FLUSH)],
                        o_bdst.at[pl.ds(dsto, FLUSH)])

    cbuf[...] = jnp.zeros((16,), _i32) + jnp.minimum(flushed + off, CAP)
    pltpu.sync_copy(cbuf, o_cnt.at[pl.ds(w * 16, 16)])


def _make_agg_body(nf):
    def body(*args):
        tables = args[:nf]
        bsrc, bdst, cnts, z16 = args[nf:nf + 4]
        outs = args[nf + 4:nf + 4 + nf]
        (acc, sbufa, dbufa, gbufa, sbufb, dbufb, gbufb, cbuf,
         sema, semb) = args[nf + 4 + nf:]
        c = lax.axis_index("c")
        s = lax.axis_index("s")
        w = c * NTILE + s
        lo = w * NPW
        pltpu.sync_copy(cnts.at[pl.ds(w * 16, 16)], cbuf)
        cnt = cbuf[...][0]
        ngr = (cnt + FLUSH - 1) // FLUSH

        for f in range(nf):
            tref = tables[f]

            def stage_fire(g, sb, db, gb, sem):
                base = w * CAP + g * FLUSH
                pltpu.sync_copy(bsrc.at[pl.ds(base, FLUSH)], sb)
                pltpu.sync_copy(bdst.at[pl.ds(base, FLUSH)],
                                db.at[pl.ds(0, FLUSH)])
                for j in range(FLUSH // CH):
                    pltpu.async_copy(tref.at[sb.at[pl.ds(j * CH, CH)]],
                                     gb.at[pl.ds(j * CH, CH)], sem)

            def wait_gb(gb, sem):
                pltpu.make_async_copy(tref.at[pl.ds(0, FLUSH)], gb,
                                      sem).wait()

            def process(g, db, gb):
                ne = jnp.clip(cnt - g * FLUSH, 0, FLUSH)
                nb = ne // 16

                def batch16(k, carry2):
                    dlv = db[pl.ds(k * 16, 16)]
                    for t in range(16):
                        dl = dlv[t]
                        plsc.addupdate(acc.at[dl], gb[k * 16 + t])
                    return carry2

                lax.fori_loop(0, nb, batch16, 0)

                def tail(e, carry2):
                    dl = db[pl.ds(e, 16)][0]
                    plsc.addupdate(acc.at[dl], gb[e])
                    return carry2

                lax.fori_loop(nb * 16, ne, tail, 0)

            pltpu.sync_copy(z16.at[pl.ds(0, NPW)], acc)

            @pl.when(ngr > 0)
            def _():
                stage_fire(0, sbufa, dbufa, gbufa, sema)

            def pair(gg, carry):
                g0 = gg * 2
                g1 = g0 + 1

                @pl.when(g0 < ngr)
                def _():
                    wait_gb(gbufa, sema)

                    @pl.when(g1 < ngr)
                    def _():
                        stage_fire(g1, sbufb, dbufb, gbufb, semb)

                    process(g0, dbufa, gbufa)

                @pl.when(g1 < ngr)
                def _():
                    wait_gb(gbufb, semb)

                    @pl.when(g1 + 1 < ngr)
                    def _():
                        stage_fire(g1 + 1, sbufa, dbufa, gbufa, sema)

                    process(g1, dbufb, gbufb)

                return carry

            lax.fori_loop(0, (ngr + 1) // 2, pair, 0)

            @pl.when(w < NW - 1)
            def _():
                pltpu.sync_copy(acc, outs[f].at[pl.ds(lo, NPW)])

            @pl.when(w == NW - 1)
            def _():
                pltpu.sync_copy(acc.at[pl.ds(0, NLAST)],
                                outs[f].at[pl.ds(lo, NLAST)])

    return body


def _sc_kernels():
    mesh = plsc.VectorSubcoreMesh(core_axis_name="c", subcore_axis_name="s")
    params = pltpu.CompilerParams(use_tc_tiling_on_sc=False,
                                  needs_layout_passes=False)
    bucket = pl.kernel(
        _bucket_body,
        out_type=(_sds((NW * CAP,), _i32), _sds((NW * CAP,), _i32),
                  _sds((NW * 16,), _i32)),
        mesh=mesh,
        scratch_types=[
            pltpu.VMEM((STAGE,), _i32),
            pltpu.VMEM((STAGE,), _i32),
            pltpu.VMEM((GRP,), _i32),
            pltpu.VMEM((GRP,), _i32),
            pltpu.VMEM((16,), _i32),
        ],
        compiler_params=params,
    )

    def agg_scratch():
        return [
            pltpu.VMEM((NPW, FB), _f32),
            pltpu.VMEM((FLUSH,), _i32),
            pltpu.VMEM((FLUSH + 16,), _i32),
            pltpu.VMEM((FLUSH, FB), _f32),
            pltpu.VMEM((FLUSH,), _i32),
            pltpu.VMEM((FLUSH + 16,), _i32),
            pltpu.VMEM((FLUSH, FB), _f32),
            pltpu.VMEM((16,), _i32),
            pltpu.SemaphoreType.DMA,
            pltpu.SemaphoreType.DMA,
        ]

    agg1 = pl.kernel(
        _make_agg_body(1),
        out_type=_sds((N, FB), _f32),
        mesh=mesh,
        scratch_types=agg_scratch(),
        compiler_params=params,
    )
    agg4 = pl.kernel(
        _make_agg_body(NFB),
        out_type=tuple(_sds((N, FB), _f32) for _ in range(NFB)),
        mesh=mesh,
        scratch_types=agg_scratch(),
        compiler_params=params,
    )
    return bucket, agg1, agg4


def _mm1_body(ap_ref, xp_ref, wl_ref, bl_ref, wr_ref, o0, o1, o2, o3, dg_ref):
    a = ap_ref[...]
    dc = jnp.maximum(a[:, 4:5], 1.0)
    aggx = a[:, 0:4] / dc
    xb = xp_ref[:, 0:4]
    y = (jnp.dot(aggx, wl_ref[...], preferred_element_type=_f32)
         + bl_ref[...]
         + jnp.dot(xb, wr_ref[...], preferred_element_type=_f32))
    y = jnp.maximum(y, 0.0)
    o0[...] = y[:, 0:16]
    o1[...] = y[:, 16:32]
    o2[...] = y[:, 32:48]
    o3[...] = y[:, 48:64]
    dg_ref[...] = jnp.broadcast_to(dc, (BN, FB))


_mm1 = pl.pallas_call(
    _mm1_body,
    grid=(NB,),
    in_specs=[
        pl.BlockSpec((BN, FB), lambda i: (i, 0)),
        pl.BlockSpec((BN, FB), lambda i: (i, 0)),
        pl.BlockSpec((4, H), lambda i: (0, 0)),
        pl.BlockSpec((1, H), lambda i: (0, 0)),
        pl.BlockSpec((4, H), lambda i: (0, 0)),
    ],
    out_specs=[pl.BlockSpec((BN, FB), lambda i: (i, 0)) for _ in range(5)],
    out_shape=[_sds((N, FB), _f32) for _ in range(5)],
)


def _mm2_body(a0, a1, a2, a3, rd, h0, h1, h2, h3, wl_ref, bl_ref, wr_ref,
              o0, o1, o2, o3):
    r = rd[...]
    agg = jnp.concatenate(
        [a0[...] / r, a1[...] / r, a2[...] / r, a3[...] / r], axis=1)
    h = jnp.concatenate([h0[...], h1[...], h2[...], h3[...]], axis=1)
    y = (jnp.dot(agg, wl_ref[...], preferred_element_type=_f32)
         + bl_ref[...]
         + jnp.dot(h, wr_ref[...], preferred_element_type=_f32))
    y = jnp.maximum(y, 0.0)
    o0[...] = y[:, 0:16]
    o1[...] = y[:, 16:32]
    o2[...] = y[:, 32:48]
    o3[...] = y[:, 48:64]


_mm2 = pl.pallas_call(
    _mm2_body,
    grid=(NB,),
    in_specs=(
        [pl.BlockSpec((BN, FB), lambda i: (i, 0)) for _ in range(9)]
        + [pl.BlockSpec((H, H), lambda i: (0, 0)),
           pl.BlockSpec((1, H), lambda i: (0, 0)),
           pl.BlockSpec((H, H), lambda i: (0, 0))]
    ),
    out_specs=[pl.BlockSpec((BN, FB), lambda i: (i, 0)) for _ in range(4)],
    out_shape=[_sds((N, FB), _f32) for _ in range(4)],
)


def _mm3_body(a0, a1, a2, a3, rd, h0, h1, h2, h3, wl_ref, bl_ref, wr_ref,
              b3_ref, out_ref):
    i = pl.program_id(0)
    r = rd[...]
    agg = jnp.concatenate(
        [a0[...] / r, a1[...] / r, a2[...] / r, a3[...] / r], axis=1)
    h = jnp.concatenate([h0[...], h1[...], h2[...], h3[...]], axis=1)
    y = (jnp.dot(agg, wl_ref[...], preferred_element_type=_f32)
         + bl_ref[...]
         + jnp.dot(h, wr_ref[...], preferred_element_type=_f32))
    seg = b3_ref[0, 0]
    oneh = jnp.equal(lax.broadcasted_iota(jnp.int32, (G, BN), 0),
                     seg[None, :]).astype(_f32)
    pooled = jnp.dot(oneh, y, preferred_element_type=_f32,
                     precision=lax.Precision.HIGHEST)
    cnt = jnp.sum(oneh, axis=1, keepdims=True)
    chunk = jnp.concatenate(
        [pooled, cnt, jnp.zeros((G, 128 - H - 1), _f32)], axis=1)

    @pl.when(i == 0)
    def _():
        out_ref[...] = jnp.zeros_like(out_ref)

    out_ref[...] += chunk


_mm3 = pl.pallas_call(
    _mm3_body,
    grid=(NB,),
    in_specs=(
        [pl.BlockSpec((BN, FB), lambda i: (i, 0)) for _ in range(9)]
        + [pl.BlockSpec((H, H), lambda i: (0, 0)),
           pl.BlockSpec((1, H), lambda i: (0, 0)),
           pl.BlockSpec((H, H), lambda i: (0, 0)),
           pl.BlockSpec((1, 1, BN), lambda i: (i, 0, 0))]
    ),
    out_specs=pl.BlockSpec((G, 128), lambda i: (0, 0)),
    out_shape=_sds((G, 128), _f32),
)


def _bn_relu(y, gamma, beta):
    m = jnp.mean(y, axis=0)
    v = jnp.mean((y - m) ** 2, axis=0)
    return jnp.maximum((y - m) / jnp.sqrt(v + 1e-5) * gamma + beta, 0.0)


def _head_body(pp_ref, wimp_ref, bimp_ref, goal_ref, wf1_ref, bf1_ref,
               wf2_ref, bf2_ref, wf3_ref, bf3_ref, wout_ref, bout_ref,
               g1_ref, be1_ref, g2_ref, be2_ref, g3_ref, be3_ref,
               im_ref, label_ref):
    pp = pp_ref[...]
    pooled = pp[:, 0:H] / jnp.maximum(pp[:, H:H + 1], 1.0)
    imp = jnp.dot(pooled, wimp_ref[...], preferred_element_type=_f32) \
        + bimp_ref[...]
    nrm = jnp.sqrt(jnp.sum(imp * imp, axis=1, keepdims=True))
    im = imp / jnp.maximum(nrm, 1e-12)

    iota6 = lax.broadcasted_iota(jnp.int32, (G, 6), 1)
    mx = jnp.max(im, axis=1, keepdims=True)
    sel = jnp.min(jnp.where(im == mx, iota6, 6), axis=1)
    gi = (sel + 5) % 6
    oneh = jnp.equal(iota6, gi[:, None]).astype(_f32)
    gsel = jnp.dot(oneh, goal_ref[...], preferred_element_type=_f32)

    x1 = jnp.concatenate([im, gsel], axis=1)
    y = jnp.dot(x1, wf1_ref[...], preferred_element_type=_f32) + bf1_ref[...]
    y = _bn_relu(y, g1_ref[...], be1_ref[...])
    y = jnp.dot(y, wf2_ref[...], preferred_element_type=_f32) + bf2_ref[...]
    y = _bn_relu(y, g2_ref[...], be2_ref[...])
    y = jnp.dot(y, wf3_ref[...], preferred_element_type=_f32) + bf3_ref[...]
    y = _bn_relu(y, g3_ref[...], be3_ref[...])
    label_ref[...] = jnp.dot(y, wout_ref[...],
                             preferred_element_type=_f32) + bout_ref[...]
    im_ref[...] = im


_head = pl.pallas_call(
    _head_body,
    out_shape=[_sds((G, 6), _f32), _sds((G, 5), _f32)],
)


def kernel(x, edge_index, batch, goal, W1l, b1l, W1r, W2l, b2l, W2r,
           W3l, b3l, W3r, Wimp, bimp, Wf1, bf1, Wf2, bf2, Wf3, bf3,
           Wout, bout, g1, be1, g2, be2, g3, be3):
    src1 = edge_index[0]
    dst1 = edge_index[1]
    xpad = jnp.concatenate(
        [x, jnp.ones((N, 1), _f32), jnp.zeros((N, FB - 5), _f32)], axis=1)
    z16 = jnp.zeros((N, FB), _f32)

    _bucket, _agg1, _agg4 = _sc_kernels()
    bsrc, bdst, cnts = _bucket(src1, dst1)

    agg1 = _agg1(xpad, bsrc, bdst, cnts, z16)
    *h1s, dg = _mm1(agg1, xpad, W1l, b1l.reshape(1, H), W1r)

    a2 = _agg4(*h1s, bsrc, bdst, cnts, z16)
    h2s = _mm2(*a2, dg, *h1s, W2l, b2l.reshape(1, H), W2r)

    a3 = _agg4(*h2s, bsrc, bdst, cnts, z16)
    pooled_plus = _mm3(*a3, dg, *h2s, W3l, b3l.reshape(1, H), W3r,
                       batch.reshape(NB, 1, BN))

    im, label = _head(pooled_plus, Wimp, bimp.reshape(1, 6), goal,
                      Wf1, bf1.reshape(1, 512), Wf2, bf2.reshape(1, 128),
                      Wf3, bf3.reshape(1, H), Wout, bout.reshape(1, 5),
                      g1.reshape(1, 512), be1.reshape(1, 512),
                      g2.reshape(1, 128), be2.reshape(1, 128),
                      g3.reshape(1, H), be3.reshape(1, H))
    return (im, label)
